# Initial kernel scaffold; baseline (speedup 1.0000x reference)
import jax
import jax.numpy as jnp
from jax.experimental import pallas as pl


def kernel(x_app, x_user, src_clicks, dst_clicks, ew_clicks, src_cb, dst_cb, ew_cb, Wn_c1, Ws_c1, b_c1, Wn_b1, Ws_b1, b_b1, Wn_c2, Ws_c2, b_c2, Wn_b2, Ws_b2, b_b2, Wc, bc):
    raise NotImplementedError("write your pallas kernel here")


# speedup vs baseline: 1.9637x; 1.9637x over previous
import functools

import jax
import jax.numpy as jnp
from jax import lax
from jax.experimental import pallas as pl
from jax.experimental.pallas import tpu as pltpu
from jax.experimental.pallas import tpu_sc as plsc

F32 = jnp.float32
I32 = jnp.int32

_EB = 128
_NPAD = 25088
_RPS = _NPAD // 16
_WCH = 112
_NWCH = _RPS // _WCH


def _mesh():
    return plsc.VectorSubcoreMesh(core_axis_name="c", subcore_axis_name="s")


def _wsum(xf, src, dst, ew, n_chunks):
    nfull, nextra = divmod(n_chunks, 16)

    @functools.partial(
        pl.kernel,
        mesh=_mesh(),
        compiler_params=pltpu.CompilerParams(use_tc_tiling_on_sc=False),
        out_type=jax.ShapeDtypeStruct((2, _NPAD, 64), F32),
        scratch_types=[
            pltpu.VMEM_SHARED((_NPAD, 64), F32),
            pltpu.VMEM((_EB,), I32),
            pltpu.VMEM((1, _EB), I32),
            pltpu.VMEM((_EB,), F32),
            pltpu.VMEM((_EB,), I32),
            pltpu.VMEM((_EB, 64), F32),
            pltpu.SemaphoreType.DMA,
        ],
    )
    def k(xf_h, src_h, dst_h, ew_h, out_h, acc, srcv, dstv, ewv, idxv, rows, sem):
        cid = lax.axis_index("c")
        sid = lax.axis_index("s")
        z16 = jnp.zeros((16,), F32)

        def zfill(r, carry):
            for kk in range(4):
                rows[r, pl.ds(kk * 16, 16)] = z16
            return carry

        lax.fori_loop(0, _EB, zfill, 0)
        rbase = sid * _RPS

        def zcopy(j, carry):
            pltpu.sync_copy(rows.at[pl.ds(0, _WCH)],
                            acc.at[pl.ds(rbase + j * _WCH, _WCH)])
            return carry

        lax.fori_loop(0, _NWCH, zcopy, 0)
        plsc.subcore_barrier()

        nloc = nfull + jnp.where(sid < nextra, 1, 0)

        def step(i, carry):
            base = (i * 16 + sid) * _EB
            pltpu.sync_copy(src_h.at[pl.ds(base, _EB)], srcv)
            pltpu.sync_copy(dst_h.at[pl.ds(base, _EB)], dstv.at[0])
            pltpu.sync_copy(ew_h.at[pl.ds(base, _EB)], ewv)
            for kk in range(8):
                idxv[pl.ds(kk * 16, 16)] = srcv[pl.ds(kk * 16, 16)] * 2 + cid
            pltpu.async_copy(xf_h.at[idxv], rows, sem).wait()

            def scale(g, c2):
                wv = ewv[pl.ds(g * 16, 16)]
                for j in range(16):
                    e = g * 16 + j
                    w = wv[j]
                    for kk in range(4):
                        rows[e, pl.ds(kk * 16, 16)] = rows[e, pl.ds(kk * 16, 16)] * w
                return c2

            lax.fori_loop(0, _EB // 16, scale, 0)
            pltpu.sync_copy(rows, acc.at[dstv.at[0]], add=True)
            return carry

        lax.fori_loop(0, nloc, step, 0)
        plsc.subcore_barrier()

        def wout(j, carry):
            r0 = rbase + j * _WCH
            pltpu.sync_copy(acc.at[pl.ds(r0, _WCH)], rows.at[pl.ds(0, _WCH)])
            pltpu.sync_copy(rows.at[pl.ds(0, _WCH)], out_h.at[cid, pl.ds(r0, _WCH)])
            return carry

        lax.fori_loop(0, _NWCH, wout, 0)

    return k(xf, src, dst, ew)


def _counts(dsts, n_chunks):
    nfull, nextra = divmod(n_chunks, 16)

    @functools.partial(
        pl.kernel,
        mesh=_mesh(),
        compiler_params=pltpu.CompilerParams(use_tc_tiling_on_sc=False),
        out_type=jax.ShapeDtypeStruct((2, _NPAD, 16), F32),
        scratch_types=[
            pltpu.VMEM_SHARED((_NPAD, 16), F32),
            pltpu.VMEM((1, _EB), I32),
            pltpu.VMEM((_EB, 16), F32),
            pltpu.VMEM((_WCH, 16), F32),
        ],
    )
    def k(dsts_h, out_h, acc, dstv, ones_v, stg):
        cid = lax.axis_index("c")
        sid = lax.axis_index("s")
        one16 = jnp.ones((16,), F32)
        z16 = jnp.zeros((16,), F32)

        def fill1(r, carry):
            ones_v[r, pl.ds(0, 16)] = one16
            return carry

        lax.fori_loop(0, _EB, fill1, 0)

        def fill0(r, carry):
            stg[r, pl.ds(0, 16)] = z16
            return carry

        lax.fori_loop(0, _WCH, fill0, 0)
        rbase = sid * _RPS

        def zcopy(j, carry):
            pltpu.sync_copy(stg, acc.at[pl.ds(rbase + j * _WCH, _WCH)])
            return carry

        lax.fori_loop(0, _NWCH, zcopy, 0)
        plsc.subcore_barrier()

        nloc = nfull + jnp.where(sid < nextra, 1, 0)

        def step(i, carry):
            base = (i * 16 + sid) * _EB
            pltpu.sync_copy(dsts_h.at[cid, pl.ds(base, _EB)], dstv.at[0])
            pltpu.sync_copy(ones_v, acc.at[dstv.at[0]], add=True)
            return carry

        lax.fori_loop(0, nloc, step, 0)
        plsc.subcore_barrier()

        def wout(j, carry):
            r0 = rbase + j * _WCH
            pltpu.sync_copy(acc.at[pl.ds(r0, _WCH)], stg)
            pltpu.sync_copy(stg, out_h.at[cid, pl.ds(r0, _WCH)])
            return carry

        lax.fori_loop(0, _NWCH, wout, 0)

    return k(dsts)


def _sage_tc(s, cnt, x, Wn, Ws, b, relu, Wc=None, bc=None):
    n = s.shape[0]
    blk = 1000
    dh = Wn.shape[1]
    dout = dh if Wc is None else Wc.shape[1]

    def body(s_ref, c_ref, x_ref, wn_ref, ws_ref, b_ref, *rest):
        if Wc is None:
            o_ref = rest[0]
        else:
            wc_ref, bc_ref, o_ref = rest
        mean = s_ref[...] / jnp.maximum(c_ref[...], 1.0)
        h = (jnp.dot(mean, wn_ref[...], preferred_element_type=F32)
             + jnp.dot(x_ref[...], ws_ref[...], preferred_element_type=F32)
             + b_ref[...])
        if relu:
            h = jnp.maximum(h, 0.0)
        if Wc is not None:
            h = jnp.dot(h, wc_ref[...], preferred_element_type=F32) + bc_ref[...]
        o_ref[...] = h

    in_specs = [
        pl.BlockSpec((blk, 128), lambda i: (i, 0)),
        pl.BlockSpec((blk, 1), lambda i: (i, 0)),
        pl.BlockSpec((blk, 128), lambda i: (i, 0)),
        pl.BlockSpec((128, dh), lambda i: (0, 0)),
        pl.BlockSpec((128, dh), lambda i: (0, 0)),
        pl.BlockSpec((1, dh), lambda i: (0, 0)),
    ]
    args = [s, cnt, x, Wn, Ws, b.reshape(1, -1)]
    if Wc is not None:
        in_specs += [pl.BlockSpec((dh, dout), lambda i: (0, 0)),
                     pl.BlockSpec((1, dout), lambda i: (0, 0))]
        args += [Wc, bc.reshape(1, -1)]
    return pl.pallas_call(
        body,
        grid=(n // blk,),
        in_specs=in_specs,
        out_specs=pl.BlockSpec((blk, dout), lambda i: (i, 0)),
        out_shape=jax.ShapeDtypeStruct((n, dout), F32),
    )(*args)


def _defold(s2, n):
    return s2[:, :n].transpose(1, 0, 2).reshape(n, 128)


def kernel(x_app, x_user, src_clicks, dst_clicks, ew_clicks, src_cb, dst_cb, ew_cb,
           Wn_c1, Ws_c1, b_c1, Wn_b1, Ws_b1, b_b1,
           Wn_c2, Ws_c2, b_c2, Wn_b2, Ws_b2, b_b2, Wc, bc):
    n_app = x_app.shape[0]
    n_user = x_user.shape[0]
    e = src_clicks.shape[0]
    n_chunks = e // _EB

    cnts = _counts(jnp.stack([dst_clicks, dst_cb]), n_chunks)
    cnt_app = cnts[0, :n_app, :1]
    cnt_user = cnts[1, :n_user, :1]

    s1a = _wsum(x_user.reshape(-1, 64), src_clicks, dst_clicks, ew_clicks, n_chunks)
    s1u = _wsum(x_app.reshape(-1, 64), src_cb, dst_cb, ew_cb, n_chunks)

    h_app = _sage_tc(_defold(s1a, n_app), cnt_app, x_app, Wn_c1, Ws_c1, b_c1, relu=True)
    h_user = _sage_tc(_defold(s1u, n_user), cnt_user, x_user, Wn_b1, Ws_b1, b_b1, relu=True)

    s2a = _wsum(h_user.reshape(-1, 64), src_clicks, dst_clicks, ew_clicks, n_chunks)
    return _sage_tc(_defold(s2a, n_app), cnt_app, h_app, Wn_c2, Ws_c2, b_c2,
                    relu=False, Wc=Wc, bc=bc)


# speedup vs baseline: 5.1990x; 2.6476x over previous
import functools

import jax
import jax.numpy as jnp
from jax import lax
from jax.experimental import pallas as pl
from jax.experimental.pallas import tpu as pltpu
from jax.experimental.pallas import tpu_sc as plsc

F32 = jnp.float32
I32 = jnp.int32

_EB = 128
_NPAD = 25088
_RPS = _NPAD // 16
_WCH = 112
_NWCH = _RPS // _WCH


def _mesh():
    return plsc.VectorSubcoreMesh(core_axis_name="c", subcore_axis_name="s")


_CPS = 196
_EPS = _CPS * _EB
_EPAD = 16 * _EPS
_SC = 14
_SE = _SC * _EB
_NSUP = _CPS // _SC


def _wsum(xf, src, dst2d, ew):
    @functools.partial(
        pl.kernel,
        mesh=_mesh(),
        compiler_params=pltpu.CompilerParams(use_tc_tiling_on_sc=False),
        out_type=jax.ShapeDtypeStruct((2, _NPAD, 64), F32),
        scratch_types=[
            pltpu.VMEM_SHARED((_NPAD, 64), F32),
            pltpu.VMEM((_SE,), I32),
            pltpu.VMEM((_SC, _EB), I32),
            pltpu.VMEM((_SE,), F32),
            pltpu.VMEM((_SE,), I32),
            pltpu.VMEM((_SC, _EB), I32),
            pltpu.VMEM((_SE,), F32),
            pltpu.VMEM((_EB,), I32),
            pltpu.VMEM((_EB,), I32),
            pltpu.VMEM((_EB, 64), F32),
            pltpu.VMEM((_EB, 64), F32),
            pltpu.SemaphoreType.DMA,
            pltpu.SemaphoreType.DMA,
            pltpu.SemaphoreType.DMA,
            pltpu.SemaphoreType.DMA,
            pltpu.SemaphoreType.DMA,
            pltpu.SemaphoreType.DMA,
        ],
    )
    def k(xf_h, src_h, dst_h, ew_h, out_h, acc,
          sA, dA, wA, sB, dB, wB, idx0, idx1, rows0, rows1,
          gs0, gs1, ss0, ss1, lsA, lsB):
        cid = lax.axis_index("c")
        sid = lax.axis_index("s")
        z16 = jnp.zeros((16,), F32)

        def zfill(r, carry):
            for kk in range(4):
                rows0[r, pl.ds(kk * 16, 16)] = z16
            return carry

        lax.fori_loop(0, _EB, zfill, 0)
        rbase = sid * _RPS

        def zcopy(j, carry):
            pltpu.sync_copy(rows0.at[pl.ds(0, _WCH)],
                            acc.at[pl.ds(rbase + j * _WCH, _WCH)])
            return carry

        lax.fori_loop(0, _NWCH, zcopy, 0)
        plsc.subcore_barrier()

        ebase = sid * _EPS
        cbase = sid * _CPS

        def load_super(t, sX, dX, wX, lsem):
            pltpu.async_copy(src_h.at[pl.ds(ebase + t * _SE, _SE)], sX, lsem)
            pltpu.async_copy(dst_h.at[pl.ds(cbase + t * _SC, _SC)], dX, lsem)
            pltpu.async_copy(ew_h.at[pl.ds(ebase + t * _SE, _SE)], wX, lsem)

        def wait_super(sX, dX, wX, lsem):
            pltpu.make_async_copy(src_h.at[pl.ds(0, _SE)], sX, lsem).wait()
            pltpu.make_async_copy(dst_h.at[pl.ds(0, _SC)], dX, lsem).wait()
            pltpu.make_async_copy(ew_h.at[pl.ds(0, _SE)], wX, lsem).wait()

        def process_super(sX, dX, wX):
            def mkidx(c, idxb):
                for kk in range(8):
                    idxb[pl.ds(kk * 16, 16)] = (
                        sX[pl.ds(c * _EB + kk * 16, 16)] * 2 + cid)

            def scale(c, rowsb):
                def grp(g, c2):
                    wv = wX[pl.ds(c * _EB + g * 16, 16)]
                    for j in range(16):
                        e = g * 16 + j
                        w = wv[j]
                        for kk in range(4):
                            rowsb[e, pl.ds(kk * 16, 16)] = (
                                rowsb[e, pl.ds(kk * 16, 16)] * w)
                    return c2
                lax.fori_loop(0, _EB // 16, grp, 0)

            def half(c, mine, other):
                idxb, rowsb, gsem, ssem = mine
                idxo, rowso, gsemo, ssemo = other
                @pl.when(c >= 1)
                def _():
                    pltpu.make_async_copy(rowso, acc.at[dX.at[0]], ssemo).wait()

                @pl.when(c + 1 < _SC)
                def _():
                    mkidx(c + 1, idxo)
                    pltpu.async_copy(xf_h.at[idxo], rowso, gsemo)

                pltpu.make_async_copy(xf_h.at[idxb], rowsb, gsem).wait()
                scale(c, rowsb)
                pltpu.async_copy(rowsb, acc.at[dX.at[c]], ssem, add=True)

            bufs0 = (idx0, rows0, gs0, ss0)
            bufs1 = (idx1, rows1, gs1, ss1)
            mkidx(0, idx0)
            pltpu.async_copy(xf_h.at[idx0], rows0, gs0)

            def step(i, carry):
                half(2 * i, bufs0, bufs1)
                half(2 * i + 1, bufs1, bufs0)
                return carry

            lax.fori_loop(0, _SC // 2, step, 0)
            pltpu.make_async_copy(rows1, acc.at[dX.at[0]], ss1).wait()

        load_super(0, sA, dA, wA, lsA)
        wait_super(sA, dA, wA, lsA)

        def souter(tt, carry):
            t0 = 2 * tt
            load_super(t0 + 1, sB, dB, wB, lsB)
            process_super(sA, dA, wA)
            wait_super(sB, dB, wB, lsB)

            @pl.when(t0 + 2 < _NSUP)
            def _():
                load_super(t0 + 2, sA, dA, wA, lsA)
            process_super(sB, dB, wB)

            @pl.when(t0 + 2 < _NSUP)
            def _():
                wait_super(sA, dA, wA, lsA)
            return carry

        lax.fori_loop(0, _NSUP // 2, souter, 0)
        plsc.subcore_barrier()

        def wout(j, carry):
            r0 = rbase + j * _WCH
            pltpu.sync_copy(acc.at[pl.ds(r0, _WCH)], rows0.at[pl.ds(0, _WCH)])
            pltpu.sync_copy(rows0.at[pl.ds(0, _WCH)], out_h.at[cid, pl.ds(r0, _WCH)])
            return carry

        lax.fori_loop(0, _NWCH, wout, 0)

    return k(xf, src, dst2d, ew)


def _counts(dsts, n_chunks):
    nfull, nextra = divmod(n_chunks, 16)

    @functools.partial(
        pl.kernel,
        mesh=_mesh(),
        compiler_params=pltpu.CompilerParams(use_tc_tiling_on_sc=False),
        out_type=jax.ShapeDtypeStruct((2, _NPAD, 16), F32),
        scratch_types=[
            pltpu.VMEM_SHARED((_NPAD, 16), F32),
            pltpu.VMEM((1, _EB), I32),
            pltpu.VMEM((_EB, 16), F32),
            pltpu.VMEM((_WCH, 16), F32),
        ],
    )
    def k(dsts_h, out_h, acc, dstv, ones_v, stg):
        cid = lax.axis_index("c")
        sid = lax.axis_index("s")
        one16 = jnp.ones((16,), F32)
        z16 = jnp.zeros((16,), F32)

        def fill1(r, carry):
            ones_v[r, pl.ds(0, 16)] = one16
            return carry

        lax.fori_loop(0, _EB, fill1, 0)

        def fill0(r, carry):
            stg[r, pl.ds(0, 16)] = z16
            return carry

        lax.fori_loop(0, _WCH, fill0, 0)
        rbase = sid * _RPS

        def zcopy(j, carry):
            pltpu.sync_copy(stg, acc.at[pl.ds(rbase + j * _WCH, _WCH)])
            return carry

        lax.fori_loop(0, _NWCH, zcopy, 0)
        plsc.subcore_barrier()

        nloc = nfull + jnp.where(sid < nextra, 1, 0)

        def step(i, carry):
            base = (i * 16 + sid) * _EB
            pltpu.sync_copy(dsts_h.at[cid, pl.ds(base, _EB)], dstv.at[0])
            pltpu.sync_copy(ones_v, acc.at[dstv.at[0]], add=True)
            return carry

        lax.fori_loop(0, nloc, step, 0)
        plsc.subcore_barrier()

        def wout(j, carry):
            r0 = rbase + j * _WCH
            pltpu.sync_copy(acc.at[pl.ds(r0, _WCH)], stg)
            pltpu.sync_copy(stg, out_h.at[cid, pl.ds(r0, _WCH)])
            return carry

        lax.fori_loop(0, _NWCH, wout, 0)

    return k(dsts)


def _sage_tc(s, cnt, x, Wn, Ws, b, relu, Wc=None, bc=None):
    n = s.shape[0]
    blk = 1000
    dh = Wn.shape[1]
    dout = dh if Wc is None else Wc.shape[1]

    def body(s_ref, c_ref, x_ref, wn_ref, ws_ref, b_ref, *rest):
        if Wc is None:
            o_ref = rest[0]
        else:
            wc_ref, bc_ref, o_ref = rest
        mean = s_ref[...] / jnp.maximum(c_ref[...], 1.0)
        h = (jnp.dot(mean, wn_ref[...], preferred_element_type=F32)
             + jnp.dot(x_ref[...], ws_ref[...], preferred_element_type=F32)
             + b_ref[...])
        if relu:
            h = jnp.maximum(h, 0.0)
        if Wc is not None:
            h = jnp.dot(h, wc_ref[...], preferred_element_type=F32) + bc_ref[...]
        o_ref[...] = h

    in_specs = [
        pl.BlockSpec((blk, 128), lambda i: (i, 0)),
        pl.BlockSpec((blk, 1), lambda i: (i, 0)),
        pl.BlockSpec((blk, 128), lambda i: (i, 0)),
        pl.BlockSpec((128, dh), lambda i: (0, 0)),
        pl.BlockSpec((128, dh), lambda i: (0, 0)),
        pl.BlockSpec((1, dh), lambda i: (0, 0)),
    ]
    args = [s, cnt, x, Wn, Ws, b.reshape(1, -1)]
    if Wc is not None:
        in_specs += [pl.BlockSpec((dh, dout), lambda i: (0, 0)),
                     pl.BlockSpec((1, dout), lambda i: (0, 0))]
        args += [Wc, bc.reshape(1, -1)]
    return pl.pallas_call(
        body,
        grid=(n // blk,),
        in_specs=in_specs,
        out_specs=pl.BlockSpec((blk, dout), lambda i: (i, 0)),
        out_shape=jax.ShapeDtypeStruct((n, dout), F32),
    )(*args)


def _defold(s2, n):
    return s2[:, :n].transpose(1, 0, 2).reshape(n, 128)


def kernel(x_app, x_user, src_clicks, dst_clicks, ew_clicks, src_cb, dst_cb, ew_cb,
           Wn_c1, Ws_c1, b_c1, Wn_b1, Ws_b1, b_b1,
           Wn_c2, Ws_c2, b_c2, Wn_b2, Ws_b2, b_b2, Wc, bc):
    n_app = x_app.shape[0]
    n_user = x_user.shape[0]
    e = src_clicks.shape[0]
    n_chunks = e // _EB

    cnts = _counts(jnp.stack([dst_clicks, dst_cb]), n_chunks)
    cnt_app = cnts[0, :n_app, :1]
    cnt_user = cnts[1, :n_user, :1]

    pad = _EPAD - e
    srcc = jnp.pad(src_clicks, (0, pad))
    dstc2 = jnp.pad(dst_clicks, (0, pad),
                    constant_values=_NPAD - 1).reshape(-1, _EB)
    ewc = jnp.pad(ew_clicks, (0, pad))
    srcb = jnp.pad(src_cb, (0, pad))
    dstb2 = jnp.pad(dst_cb, (0, pad),
                    constant_values=_NPAD - 1).reshape(-1, _EB)
    ewb = jnp.pad(ew_cb, (0, pad))

    s1a = _wsum(x_user.reshape(-1, 64), srcc, dstc2, ewc)
    s1u = _wsum(x_app.reshape(-1, 64), srcb, dstb2, ewb)

    h_app = _sage_tc(_defold(s1a, n_app), cnt_app, x_app, Wn_c1, Ws_c1, b_c1, relu=True)
    h_user = _sage_tc(_defold(s1u, n_user), cnt_user, x_user, Wn_b1, Ws_b1, b_b1, relu=True)

    s2a = _wsum(h_user.reshape(-1, 64), srcc, dstc2, ewc)
    return _sage_tc(_defold(s2a, n_app), cnt_app, h_app, Wn_c2, Ws_c2, b_c2,
                    relu=False, Wc=Wc, bc=bc)


# speedup vs baseline: 5.6598x; 1.0886x over previous
import functools

import jax
import jax.numpy as jnp
from jax import lax
from jax.experimental import pallas as pl
from jax.experimental.pallas import tpu as pltpu
from jax.experimental.pallas import tpu_sc as plsc

F32 = jnp.float32
I32 = jnp.int32

_EB = 128
_NPAD = 25088
_RPS = _NPAD // 16
_WCH = 112
_NWCH = _RPS // _WCH


def _mesh():
    return plsc.VectorSubcoreMesh(core_axis_name="c", subcore_axis_name="s")


_CPS = 196
_EPS = _CPS * _EB
_EPAD = 16 * _EPS
_SC = 14
_SE = _SC * _EB
_NSUP = _CPS // _SC


def _wsum(xf, src, dst2d, ew):
    @functools.partial(
        pl.kernel,
        mesh=_mesh(),
        compiler_params=pltpu.CompilerParams(use_tc_tiling_on_sc=False),
        out_type=jax.ShapeDtypeStruct((2, _NPAD, 64), F32),
        scratch_types=[
            pltpu.VMEM_SHARED((_NPAD, 64), F32),
            pltpu.VMEM((_SE,), I32),
            pltpu.VMEM((_SC, _EB), I32),
            pltpu.VMEM((_SE,), F32),
            pltpu.VMEM((_SE,), I32),
            pltpu.VMEM((_SC, _EB), I32),
            pltpu.VMEM((_SE,), F32),
            pltpu.VMEM((_EB,), I32),
            pltpu.VMEM((_EB,), I32),
            pltpu.VMEM((_EB, 64), F32),
            pltpu.VMEM((_EB, 64), F32),
            pltpu.SemaphoreType.DMA,
            pltpu.SemaphoreType.DMA,
            pltpu.SemaphoreType.DMA,
            pltpu.SemaphoreType.DMA,
            pltpu.SemaphoreType.DMA,
            pltpu.SemaphoreType.DMA,
        ],
    )
    def k(xf_h, src_h, dst_h, ew_h, out_h, acc,
          sA, dA, wA, sB, dB, wB, idx0, idx1, rows0, rows1,
          gs0, gs1, ss0, ss1, lsA, lsB):
        cid = lax.axis_index("c")
        sid = lax.axis_index("s")
        z16 = jnp.zeros((16,), F32)

        def zfill(r, carry):
            for kk in range(4):
                rows0[r, pl.ds(kk * 16, 16)] = z16
            return carry

        lax.fori_loop(0, _EB, zfill, 0)
        rbase = sid * _RPS

        def zcopy(j, carry):
            pltpu.sync_copy(rows0.at[pl.ds(0, _WCH)],
                            acc.at[pl.ds(rbase + j * _WCH, _WCH)])
            return carry

        lax.fori_loop(0, _NWCH, zcopy, 0)
        plsc.subcore_barrier()

        ebase = sid * _EPS
        cbase = sid * _CPS

        def load_super(t, sX, dX, wX, lsem):
            pltpu.async_copy(src_h.at[pl.ds(ebase + t * _SE, _SE)], sX, lsem)
            pltpu.async_copy(dst_h.at[pl.ds(cbase + t * _SC, _SC)], dX, lsem)
            pltpu.async_copy(ew_h.at[pl.ds(ebase + t * _SE, _SE)], wX, lsem)

        def wait_super(sX, dX, wX, lsem):
            pltpu.make_async_copy(src_h.at[pl.ds(0, _SE)], sX, lsem).wait()
            pltpu.make_async_copy(dst_h.at[pl.ds(0, _SC)], dX, lsem).wait()
            pltpu.make_async_copy(ew_h.at[pl.ds(0, _SE)], wX, lsem).wait()

        def process_super(sX, dX, wX):
            def mkidx(c, idxb):
                for kk in range(8):
                    idxb[pl.ds(kk * 16, 16)] = (
                        sX[pl.ds(c * _EB + kk * 16, 16)] * 2 + cid)

            def scale(c, rowsb):
                def grp(g, c2):
                    wv = wX[pl.ds(c * _EB + g * 16, 16)]
                    for j in range(16):
                        e = g * 16 + j
                        w = wv[j]
                        for kk in range(4):
                            rowsb[e, pl.ds(kk * 16, 16)] = (
                                rowsb[e, pl.ds(kk * 16, 16)] * w)
                    return c2
                lax.fori_loop(0, _EB // 16, grp, 0)

            def half(c, mine, other):
                idxb, rowsb, gsem, ssem = mine
                idxo, rowso, gsemo, ssemo = other
                @pl.when(c >= 1)
                def _():
                    pltpu.make_async_copy(rowso, acc.at[dX.at[0]], ssemo).wait()

                @pl.when(c + 1 < _SC)
                def _():
                    mkidx(c + 1, idxo)
                    pltpu.async_copy(xf_h.at[idxo], rowso, gsemo)

                pltpu.make_async_copy(xf_h.at[idxb], rowsb, gsem).wait()
                scale(c, rowsb)
                pltpu.async_copy(rowsb, acc.at[dX.at[c]], ssem, add=True)

            bufs0 = (idx0, rows0, gs0, ss0)
            bufs1 = (idx1, rows1, gs1, ss1)
            mkidx(0, idx0)
            pltpu.async_copy(xf_h.at[idx0], rows0, gs0)

            def step(i, carry):
                half(2 * i, bufs0, bufs1)
                half(2 * i + 1, bufs1, bufs0)
                return carry

            lax.fori_loop(0, _SC // 2, step, 0)
            pltpu.make_async_copy(rows1, acc.at[dX.at[0]], ss1).wait()

        load_super(0, sA, dA, wA, lsA)
        wait_super(sA, dA, wA, lsA)

        def souter(tt, carry):
            t0 = 2 * tt
            load_super(t0 + 1, sB, dB, wB, lsB)
            process_super(sA, dA, wA)
            wait_super(sB, dB, wB, lsB)

            @pl.when(t0 + 2 < _NSUP)
            def _():
                load_super(t0 + 2, sA, dA, wA, lsA)
            process_super(sB, dB, wB)

            @pl.when(t0 + 2 < _NSUP)
            def _():
                wait_super(sA, dA, wA, lsA)
            return carry

        lax.fori_loop(0, _NSUP // 2, souter, 0)
        plsc.subcore_barrier()

        def wout(j, carry):
            r0 = rbase + j * _WCH
            pltpu.sync_copy(acc.at[pl.ds(r0, _WCH)], rows0.at[pl.ds(0, _WCH)])
            pltpu.sync_copy(rows0.at[pl.ds(0, _WCH)], out_h.at[cid, pl.ds(r0, _WCH)])
            return carry

        lax.fori_loop(0, _NWCH, wout, 0)

    return k(xf, src, dst2d, ew)


def _counts(dsts2d):
    @functools.partial(
        pl.kernel,
        mesh=_mesh(),
        compiler_params=pltpu.CompilerParams(use_tc_tiling_on_sc=False),
        out_type=jax.ShapeDtypeStruct((2, _NPAD, 16), F32),
        scratch_types=[
            pltpu.VMEM_SHARED((_NPAD, 16), F32),
            pltpu.VMEM((_SC, _EB), I32),
            pltpu.VMEM((_SC, _EB), I32),
            pltpu.VMEM((_EB, 16), F32),
            pltpu.VMEM((_WCH, 16), F32),
            pltpu.SemaphoreType.DMA,
            pltpu.SemaphoreType.DMA,
            pltpu.SemaphoreType.DMA,
            pltpu.SemaphoreType.DMA,
        ],
    )
    def k(dsts_h, out_h, acc, dA, dB, ones_v, stg, ss0, ss1, lsA, lsB):
        cid = lax.axis_index("c")
        sid = lax.axis_index("s")
        one16 = jnp.ones((16,), F32)
        z16 = jnp.zeros((16,), F32)

        def fill1(r, carry):
            ones_v[r, pl.ds(0, 16)] = one16
            return carry

        lax.fori_loop(0, _EB, fill1, 0)

        def fill0(r, carry):
            stg[r, pl.ds(0, 16)] = z16
            return carry

        lax.fori_loop(0, _WCH, fill0, 0)
        rbase = sid * _RPS

        def zcopy(j, carry):
            pltpu.sync_copy(stg, acc.at[pl.ds(rbase + j * _WCH, _WCH)])
            return carry

        lax.fori_loop(0, _NWCH, zcopy, 0)
        plsc.subcore_barrier()

        cbase = sid * _CPS

        def load_super(t, dX, lsem):
            pltpu.async_copy(dsts_h.at[cid, pl.ds(cbase + t * _SC, _SC)],
                             dX, lsem)

        def wait_super(dX, lsem):
            pltpu.make_async_copy(dsts_h.at[cid, pl.ds(0, _SC)], dX,
                                  lsem).wait()

        def process_super(dX):
            def half(c, ssem, ssemo):
                @pl.when(c >= 1)
                def _():
                    pltpu.make_async_copy(ones_v, acc.at[dX.at[0]],
                                          ssemo).wait()
                pltpu.async_copy(ones_v, acc.at[dX.at[c]], ssem, add=True)

            def step(i, carry):
                half(2 * i, ss0, ss1)
                half(2 * i + 1, ss1, ss0)
                return carry

            lax.fori_loop(0, _SC // 2, step, 0)
            pltpu.make_async_copy(ones_v, acc.at[dX.at[0]], ss1).wait()

        load_super(0, dA, lsA)
        wait_super(dA, lsA)

        def souter(tt, carry):
            t0 = 2 * tt
            load_super(t0 + 1, dB, lsB)
            process_super(dA)
            wait_super(dB, lsB)

            @pl.when(t0 + 2 < _NSUP)
            def _():
                load_super(t0 + 2, dA, lsA)
            process_super(dB)

            @pl.when(t0 + 2 < _NSUP)
            def _():
                wait_super(dA, lsA)
            return carry

        lax.fori_loop(0, _NSUP // 2, souter, 0)
        plsc.subcore_barrier()

        def wout(j, carry):
            r0 = rbase + j * _WCH
            pltpu.sync_copy(acc.at[pl.ds(r0, _WCH)], stg)
            pltpu.sync_copy(stg, out_h.at[cid, pl.ds(r0, _WCH)])
            return carry

        lax.fori_loop(0, _NWCH, wout, 0)

    return k(dsts2d)


def _sage_tc(s, cnt, x, Wn, Ws, b, relu, Wc=None, bc=None):
    n = s.shape[0]
    blk = 1000
    dh = Wn.shape[1]
    dout = dh if Wc is None else Wc.shape[1]

    def body(s_ref, c_ref, x_ref, wn_ref, ws_ref, b_ref, *rest):
        if Wc is None:
            o_ref = rest[0]
        else:
            wc_ref, bc_ref, o_ref = rest
        mean = s_ref[...] / jnp.maximum(c_ref[...], 1.0)
        h = (jnp.dot(mean, wn_ref[...], preferred_element_type=F32)
             + jnp.dot(x_ref[...], ws_ref[...], preferred_element_type=F32)
             + b_ref[...])
        if relu:
            h = jnp.maximum(h, 0.0)
        if Wc is not None:
            h = jnp.dot(h, wc_ref[...], preferred_element_type=F32) + bc_ref[...]
        o_ref[...] = h

    in_specs = [
        pl.BlockSpec((blk, 128), lambda i: (i, 0)),
        pl.BlockSpec((blk, 1), lambda i: (i, 0)),
        pl.BlockSpec((blk, 128), lambda i: (i, 0)),
        pl.BlockSpec((128, dh), lambda i: (0, 0)),
        pl.BlockSpec((128, dh), lambda i: (0, 0)),
        pl.BlockSpec((1, dh), lambda i: (0, 0)),
    ]
    args = [s, cnt, x, Wn, Ws, b.reshape(1, -1)]
    if Wc is not None:
        in_specs += [pl.BlockSpec((dh, dout), lambda i: (0, 0)),
                     pl.BlockSpec((1, dout), lambda i: (0, 0))]
        args += [Wc, bc.reshape(1, -1)]
    return pl.pallas_call(
        body,
        grid=(n // blk,),
        in_specs=in_specs,
        out_specs=pl.BlockSpec((blk, dout), lambda i: (i, 0)),
        out_shape=jax.ShapeDtypeStruct((n, dout), F32),
    )(*args)


def _defold(s2, n):
    return s2[:, :n].transpose(1, 0, 2).reshape(n, 128)


def kernel(x_app, x_user, src_clicks, dst_clicks, ew_clicks, src_cb, dst_cb, ew_cb,
           Wn_c1, Ws_c1, b_c1, Wn_b1, Ws_b1, b_b1,
           Wn_c2, Ws_c2, b_c2, Wn_b2, Ws_b2, b_b2, Wc, bc):
    n_app = x_app.shape[0]
    n_user = x_user.shape[0]
    e = src_clicks.shape[0]

    pad = _EPAD - e
    srcc = jnp.pad(src_clicks, (0, pad))
    dstc2 = jnp.pad(dst_clicks, (0, pad),
                    constant_values=_NPAD - 1).reshape(-1, _EB)
    ewc = jnp.pad(ew_clicks, (0, pad))
    srcb = jnp.pad(src_cb, (0, pad))
    dstb2 = jnp.pad(dst_cb, (0, pad),
                    constant_values=_NPAD - 1).reshape(-1, _EB)
    ewb = jnp.pad(ew_cb, (0, pad))

    cnts = _counts(jnp.stack([dstc2, dstb2]))
    cnt_app = cnts[0, :n_app, :1]
    cnt_user = cnts[1, :n_user, :1]

    s1a = _wsum(x_user.reshape(-1, 64), srcc, dstc2, ewc)
    s1u = _wsum(x_app.reshape(-1, 64), srcb, dstb2, ewb)

    h_app = _sage_tc(_defold(s1a, n_app), cnt_app, x_app, Wn_c1, Ws_c1, b_c1, relu=True)
    h_user = _sage_tc(_defold(s1u, n_user), cnt_user, x_user, Wn_b1, Ws_b1, b_b1, relu=True)

    s2a = _wsum(h_user.reshape(-1, 64), srcc, dstc2, ewc)
    return _sage_tc(_defold(s2a, n_app), cnt_app, h_app, Wn_c2, Ws_c2, b_c2,
                    relu=False, Wc=Wc, bc=bc)


# speedup vs baseline: 6.2306x; 1.1008x over previous
import functools

import jax
import jax.numpy as jnp
from jax import lax
from jax.experimental import pallas as pl
from jax.experimental.pallas import tpu as pltpu
from jax.experimental.pallas import tpu_sc as plsc

F32 = jnp.float32
I32 = jnp.int32

_EB = 128
_NPAD = 25088
_RPS = _NPAD // 16
_WCH = 112
_NWCH = _RPS // _WCH


def _mesh():
    return plsc.VectorSubcoreMesh(core_axis_name="c", subcore_axis_name="s")


_CPS = 196
_EPS = _CPS * _EB
_EPAD = 16 * _EPS
_SC = 14
_SE = _SC * _EB
_NSUP = _CPS // _SC


def _wsum(xf, src, dst2d, ew):
    @functools.partial(
        pl.kernel,
        mesh=_mesh(),
        compiler_params=pltpu.CompilerParams(use_tc_tiling_on_sc=False),
        out_type=jax.ShapeDtypeStruct((2, _NPAD, 64), F32),
        scratch_types=[
            pltpu.VMEM_SHARED((_NPAD, 64), F32),
            pltpu.VMEM((_SE,), I32),
            pltpu.VMEM((_SC, _EB), I32),
            pltpu.VMEM((_SE,), F32),
            pltpu.VMEM((_SE,), I32),
            pltpu.VMEM((_SC, _EB), I32),
            pltpu.VMEM((_SE,), F32),
            pltpu.VMEM((_EB,), I32),
            pltpu.VMEM((_EB,), I32),
            pltpu.VMEM((_EB, 64), F32),
            pltpu.VMEM((_EB, 64), F32),
            pltpu.SemaphoreType.DMA,
            pltpu.SemaphoreType.DMA,
            pltpu.SemaphoreType.DMA,
            pltpu.SemaphoreType.DMA,
            pltpu.SemaphoreType.DMA,
            pltpu.SemaphoreType.DMA,
        ],
    )
    def k(xf_h, src_h, dst_h, ew_h, out_h, acc,
          sA, dA, wA, sB, dB, wB, idx0, idx1, rows0, rows1,
          gs0, gs1, ss0, ss1, lsA, lsB):
        cid = lax.axis_index("c")
        sid = lax.axis_index("s")
        z16 = jnp.zeros((16,), F32)

        def zfill(r, carry):
            for kk in range(4):
                rows0[r, pl.ds(kk * 16, 16)] = z16
            return carry

        lax.fori_loop(0, _EB, zfill, 0)
        rbase = sid * _RPS

        def zcopy(j, carry):
            pltpu.sync_copy(rows0.at[pl.ds(0, _WCH)],
                            acc.at[pl.ds(rbase + j * _WCH, _WCH)])
            return carry

        lax.fori_loop(0, _NWCH, zcopy, 0)
        plsc.subcore_barrier()

        ebase = sid * _EPS
        cbase = sid * _CPS

        def load_super(t, sX, dX, wX, lsem):
            pltpu.async_copy(src_h.at[pl.ds(ebase + t * _SE, _SE)], sX, lsem)
            pltpu.async_copy(dst_h.at[pl.ds(cbase + t * _SC, _SC)], dX, lsem)
            pltpu.async_copy(ew_h.at[pl.ds(ebase + t * _SE, _SE)], wX, lsem)

        def wait_super(sX, dX, wX, lsem):
            pltpu.make_async_copy(src_h.at[pl.ds(0, _SE)], sX, lsem).wait()
            pltpu.make_async_copy(dst_h.at[pl.ds(0, _SC)], dX, lsem).wait()
            pltpu.make_async_copy(ew_h.at[pl.ds(0, _SE)], wX, lsem).wait()

        def process_super(sX, dX, wX):
            def mkidx(c, idxb):
                for kk in range(_EB // 16):
                    idxb[pl.ds(kk * 16, 16)] = (
                        sX[pl.ds(c * _EB + kk * 16, 16)] * 2 + cid)

            def scale(c, rowsb):
                def grp(g, c2):
                    wv = wX[pl.ds(c * _EB + g * 16, 16)]
                    for j in range(16):
                        e = g * 16 + j
                        w = wv[j]
                        for kk in range(4):
                            rowsb[e, pl.ds(kk * 16, 16)] = (
                                rowsb[e, pl.ds(kk * 16, 16)] * w)
                    return c2
                lax.fori_loop(0, _EB // 16, grp, 0)

            def half(c, mine, other):
                idxb, rowsb, gsem, ssem = mine
                idxo, rowso, gsemo, ssemo = other
                @pl.when(c >= 1)
                def _():
                    pltpu.make_async_copy(rowso, acc.at[dX.at[0]], ssemo).wait()

                @pl.when(c + 1 < _SC)
                def _():
                    mkidx(c + 1, idxo)
                    pltpu.async_copy(xf_h.at[idxo], rowso, gsemo)

                pltpu.make_async_copy(xf_h.at[idxb], rowsb, gsem).wait()
                scale(c, rowsb)
                pltpu.async_copy(rowsb, acc.at[dX.at[c]], ssem, add=True)

            bufs0 = (idx0, rows0, gs0, ss0)
            bufs1 = (idx1, rows1, gs1, ss1)
            mkidx(0, idx0)
            pltpu.async_copy(xf_h.at[idx0], rows0, gs0)

            def step(i, carry):
                half(2 * i, bufs0, bufs1)
                half(2 * i + 1, bufs1, bufs0)
                return carry

            lax.fori_loop(0, _SC // 2, step, 0)
            pltpu.make_async_copy(rows1, acc.at[dX.at[0]], ss1).wait()

        load_super(0, sA, dA, wA, lsA)
        wait_super(sA, dA, wA, lsA)

        def souter(tt, carry):
            t0 = 2 * tt
            load_super(t0 + 1, sB, dB, wB, lsB)
            process_super(sA, dA, wA)
            wait_super(sB, dB, wB, lsB)

            @pl.when(t0 + 2 < _NSUP)
            def _():
                load_super(t0 + 2, sA, dA, wA, lsA)
            process_super(sB, dB, wB)

            @pl.when(t0 + 2 < _NSUP)
            def _():
                wait_super(sA, dA, wA, lsA)
            return carry

        lax.fori_loop(0, _NSUP // 2, souter, 0)
        plsc.subcore_barrier()

        def wout(j, carry):
            r0 = rbase + j * _WCH
            pltpu.sync_copy(acc.at[pl.ds(r0, _WCH)], rows0.at[pl.ds(0, _WCH)])
            pltpu.sync_copy(rows0.at[pl.ds(0, _WCH)], out_h.at[cid, pl.ds(r0, _WCH)])
            return carry

        lax.fori_loop(0, _NWCH, wout, 0)

    return k(xf, src, dst2d, ew)


def _counts(dsts2d):
    @functools.partial(
        pl.kernel,
        mesh=_mesh(),
        compiler_params=pltpu.CompilerParams(use_tc_tiling_on_sc=False),
        out_type=jax.ShapeDtypeStruct((2, _NPAD, 16), F32),
        scratch_types=[
            pltpu.VMEM_SHARED((_NPAD, 16), F32),
            pltpu.VMEM((_SC, _EB), I32),
            pltpu.VMEM((_SC, _EB), I32),
            pltpu.VMEM((_EB, 16), F32),
            pltpu.VMEM((_WCH, 16), F32),
            pltpu.SemaphoreType.DMA,
            pltpu.SemaphoreType.DMA,
            pltpu.SemaphoreType.DMA,
            pltpu.SemaphoreType.DMA,
        ],
    )
    def k(dsts_h, out_h, acc, dA, dB, ones_v, stg, ss0, ss1, lsA, lsB):
        cid = lax.axis_index("c")
        sid = lax.axis_index("s")
        one16 = jnp.ones((16,), F32)
        z16 = jnp.zeros((16,), F32)

        def fill1(r, carry):
            ones_v[r, pl.ds(0, 16)] = one16
            return carry

        lax.fori_loop(0, _EB, fill1, 0)

        def fill0(r, carry):
            stg[r, pl.ds(0, 16)] = z16
            return carry

        lax.fori_loop(0, _WCH, fill0, 0)
        rbase = sid * _RPS

        def zcopy(j, carry):
            pltpu.sync_copy(stg, acc.at[pl.ds(rbase + j * _WCH, _WCH)])
            return carry

        lax.fori_loop(0, _NWCH, zcopy, 0)
        plsc.subcore_barrier()

        cbase = sid * _CPS

        def load_super(t, dX, lsem):
            pltpu.async_copy(dsts_h.at[cid, pl.ds(cbase + t * _SC, _SC)],
                             dX, lsem)

        def wait_super(dX, lsem):
            pltpu.make_async_copy(dsts_h.at[cid, pl.ds(0, _SC)], dX,
                                  lsem).wait()

        def process_super(dX):
            def half(c, ssem, ssemo):
                @pl.when(c >= 1)
                def _():
                    pltpu.make_async_copy(ones_v, acc.at[dX.at[0]],
                                          ssemo).wait()
                pltpu.async_copy(ones_v, acc.at[dX.at[c]], ssem, add=True)

            def step(i, carry):
                half(2 * i, ss0, ss1)
                half(2 * i + 1, ss1, ss0)
                return carry

            lax.fori_loop(0, _SC // 2, step, 0)
            pltpu.make_async_copy(ones_v, acc.at[dX.at[0]], ss1).wait()

        load_super(0, dA, lsA)
        wait_super(dA, lsA)

        def souter(tt, carry):
            t0 = 2 * tt
            load_super(t0 + 1, dB, lsB)
            process_super(dA)
            wait_super(dB, lsB)

            @pl.when(t0 + 2 < _NSUP)
            def _():
                load_super(t0 + 2, dA, lsA)
            process_super(dB)

            @pl.when(t0 + 2 < _NSUP)
            def _():
                wait_super(dA, lsA)
            return carry

        lax.fori_loop(0, _NSUP // 2, souter, 0)
        plsc.subcore_barrier()

        def wout(j, carry):
            r0 = rbase + j * _WCH
            pltpu.sync_copy(acc.at[pl.ds(r0, _WCH)], stg)
            pltpu.sync_copy(stg, out_h.at[cid, pl.ds(r0, _WCH)])
            return carry

        lax.fori_loop(0, _NWCH, wout, 0)

    return k(dsts2d)


def _sage_tc(s2, cnt, x, Wn, Ws, b, relu, Wc=None, bc=None):
    n = x.shape[0]
    blk = 1000
    dh = Wn.shape[1]
    dout = dh if Wc is None else Wc.shape[1]

    def body(s_ref, c_ref, x_ref, wn_ref, ws_ref, b_ref, *rest):
        if Wc is None:
            o_ref = rest[0]
        else:
            wc_ref, bc_ref, o_ref = rest
        inv = 1.0 / jnp.maximum(c_ref[...], 1.0)
        wn = wn_ref[...]
        h = (jnp.dot(s_ref[0] * inv, wn[:64], preferred_element_type=F32)
             + jnp.dot(s_ref[1] * inv, wn[64:], preferred_element_type=F32)
             + jnp.dot(x_ref[...], ws_ref[...], preferred_element_type=F32)
             + b_ref[...])
        if relu:
            h = jnp.maximum(h, 0.0)
        if Wc is not None:
            h = jnp.dot(h, wc_ref[...], preferred_element_type=F32) + bc_ref[...]
        o_ref[...] = h

    in_specs = [
        pl.BlockSpec((2, blk, 64), lambda i: (0, i, 0)),
        pl.BlockSpec((blk, 1), lambda i: (i, 0)),
        pl.BlockSpec((blk, 128), lambda i: (i, 0)),
        pl.BlockSpec((128, dh), lambda i: (0, 0)),
        pl.BlockSpec((128, dh), lambda i: (0, 0)),
        pl.BlockSpec((1, dh), lambda i: (0, 0)),
    ]
    args = [s2, cnt, x, Wn, Ws, b.reshape(1, -1)]
    if Wc is not None:
        in_specs += [pl.BlockSpec((dh, dout), lambda i: (0, 0)),
                     pl.BlockSpec((1, dout), lambda i: (0, 0))]
        args += [Wc, bc.reshape(1, -1)]
    return pl.pallas_call(
        body,
        grid=(n // blk,),
        in_specs=in_specs,
        out_specs=pl.BlockSpec((blk, dout), lambda i: (i, 0)),
        out_shape=jax.ShapeDtypeStruct((n, dout), F32),
    )(*args)


def kernel(x_app, x_user, src_clicks, dst_clicks, ew_clicks, src_cb, dst_cb, ew_cb,
           Wn_c1, Ws_c1, b_c1, Wn_b1, Ws_b1, b_b1,
           Wn_c2, Ws_c2, b_c2, Wn_b2, Ws_b2, b_b2, Wc, bc):
    n_app = x_app.shape[0]
    n_user = x_user.shape[0]
    e = src_clicks.shape[0]

    pad = _EPAD - e
    srcc = jnp.pad(src_clicks, (0, pad))
    dstc2 = jnp.pad(dst_clicks, (0, pad),
                    constant_values=_NPAD - 1).reshape(-1, _EB)
    ewc = jnp.pad(ew_clicks, (0, pad))
    srcb = jnp.pad(src_cb, (0, pad))
    dstb2 = jnp.pad(dst_cb, (0, pad),
                    constant_values=_NPAD - 1).reshape(-1, _EB)
    ewb = jnp.pad(ew_cb, (0, pad))

    cnts = _counts(jnp.stack([dstc2, dstb2]))
    cnt_app = cnts[0, :n_app, :1]
    cnt_user = cnts[1, :n_user, :1]

    s1a = _wsum(x_user.reshape(-1, 64), srcc, dstc2, ewc)
    s1u = _wsum(x_app.reshape(-1, 64), srcb, dstb2, ewb)

    h_app = _sage_tc(s1a, cnt_app, x_app, Wn_c1, Ws_c1, b_c1, relu=True)
    h_user = _sage_tc(s1u, cnt_user, x_user, Wn_b1, Ws_b1, b_b1, relu=True)

    s2a = _wsum(h_user.reshape(-1, 64), srcc, dstc2, ewc)
    return _sage_tc(s2a, cnt_app, h_app, Wn_c2, Ws_c2, b_c2,
                    relu=False, Wc=Wc, bc=bc)
